# Initial kernel scaffold; baseline (speedup 1.0000x reference)
import jax
import jax.numpy as jnp
from jax.experimental import pallas as pl


def kernel(x, edge_index, edge_weight, W, b, k):
    raise NotImplementedError("write your pallas kernel here")


# speedup vs baseline: 3.4423x; 3.4423x over previous
import functools

import jax
import jax.numpy as jnp
from jax import lax
from jax.experimental import pallas as pl
from jax.experimental.pallas import tpu as pltpu
from jax.experimental.pallas import tpu_sc as plsc

_C = 128
_NUM_WORKERS = 32


def _sc_spmm(h, srcp, dstp, wp, zeros, n_nodes, d, chunks_per_worker):
    mesh = plsc.VectorSubcoreMesh(core_axis_name="c", subcore_axis_name="s")
    rows_per_sub = n_nodes // 16
    nvec = d // 16

    @functools.partial(
        pl.kernel,
        out_type=(
            jax.ShapeDtypeStruct((n_nodes, d), jnp.float32),
            jax.ShapeDtypeStruct((n_nodes, d), jnp.float32),
        ),
        mesh=mesh,
        scratch_types=[
            pltpu.VMEM((_C,), jnp.int32),
            pltpu.VMEM((_C,), jnp.int32),
            pltpu.VMEM((_C,), jnp.float32),
            pltpu.VMEM((_C, d), jnp.float32),
            pltpu.VMEM_SHARED((n_nodes, d), jnp.float32),
            pltpu.SemaphoreType.DMA,
        ],
    )
    def spmm(h_hbm, src_hbm, dst_hbm, w_hbm, z_hbm, p0_hbm, p1_hbm,
             sidx, didx, wv, rows, acc, sem):
        cid = lax.axis_index("c")
        sid = lax.axis_index("s")
        wid = sid * 2 + cid

        pltpu.sync_copy(z_hbm.at[pl.ds(sid * rows_per_sub, rows_per_sub)],
                        acc.at[pl.ds(sid * rows_per_sub, rows_per_sub)])
        plsc.subcore_barrier()

        base0 = wid * (chunks_per_worker * _C)

        def chunk_body(j, carry):
            base = base0 + j * _C
            pltpu.sync_copy(src_hbm.at[pl.ds(base, _C)], sidx)
            pltpu.sync_copy(dst_hbm.at[pl.ds(base, _C)], didx)
            pltpu.sync_copy(w_hbm.at[pl.ds(base, _C)], wv)
            pltpu.async_copy(h_hbm.at[sidx], rows, sem).wait()

            def grp_body(g, c2):
                w16 = wv[pl.ds(g * 16, 16)]
                r0 = g * 16
                for j in range(16):
                    wr = w16[j]
                    for v in range(nvec):
                        sl = pl.ds(v * 16, 16)
                        rows[r0 + j, sl] = rows[r0 + j, sl] * wr
                return c2

            lax.fori_loop(0, _C // 16, grp_body, 0)

            pltpu.sync_copy(rows, acc.at[didx], add=True)
            return carry

        lax.fori_loop(0, chunks_per_worker, chunk_body, 0)
        plsc.subcore_barrier()

        sl = pl.ds(sid * rows_per_sub, rows_per_sub)

        @pl.when(cid == 0)
        def _():
            pltpu.sync_copy(acc.at[sl], p0_hbm.at[sl])

        @pl.when(cid == 1)
        def _():
            pltpu.sync_copy(acc.at[sl], p1_hbm.at[sl])

    return spmm(h, srcp, dstp, wp, zeros)


def _tc_combine(a, b):
    n, d = a.shape
    br = next(c for c in (1280, 1024, 640, 512, 256, 128, 8) if n % c == 0)

    def body(a_ref, b_ref, o_ref):
        o_ref[...] = a_ref[...] + b_ref[...]

    return pl.pallas_call(
        body,
        out_shape=jax.ShapeDtypeStruct((n, d), jnp.float32),
        grid=(n // br,),
        in_specs=[pl.BlockSpec((br, d), lambda i: (i, 0)),
                  pl.BlockSpec((br, d), lambda i: (i, 0))],
        out_specs=pl.BlockSpec((br, d), lambda i: (i, 0)),
    )(a, b)


def _tc_linear(h, wt, bias):
    n, d = h.shape
    br = 1000

    def body(h_ref, w_ref, b_ref, o_ref):
        o_ref[...] = jnp.dot(h_ref[...], w_ref[...],
                             preferred_element_type=jnp.float32) + b_ref[...]

    return pl.pallas_call(
        body,
        out_shape=jax.ShapeDtypeStruct((n, d), jnp.float32),
        grid=(n // br,),
        in_specs=[pl.BlockSpec((br, d), lambda i: (i, 0)),
                  pl.BlockSpec((d, d), lambda i: (0, 0)),
                  pl.BlockSpec((1, d), lambda i: (0, 0))],
        out_specs=pl.BlockSpec((br, d), lambda i: (i, 0)),
    )(h, wt, bias.reshape(1, d))


def kernel(x, edge_index, edge_weight, W, b, k):
    n_nodes, d = x.shape
    e = edge_weight.shape[0]
    dst = edge_index[0]
    src = edge_index[1]

    chunks_per_worker = -(-e // (_NUM_WORKERS * _C))
    e_pad = _NUM_WORKERS * chunks_per_worker * _C
    pad = e_pad - e
    srcp = jnp.concatenate([src, jnp.zeros((pad,), jnp.int32)])
    dstp = jnp.concatenate([dst, jnp.zeros((pad,), jnp.int32)])
    wp = jnp.concatenate([edge_weight, jnp.zeros((pad,), jnp.float32)])

    n_pad = -(-n_nodes // 128) * 128
    xp = jnp.pad(x, ((0, n_pad - n_nodes), (0, 0)))
    zeros = jnp.zeros((n_pad, d), jnp.float32)

    def body(_, h):
        p0, p1 = _sc_spmm(h, srcp, dstp, wp, zeros, n_pad, d,
                          chunks_per_worker)
        return _tc_combine(p0, p1)

    h = lax.fori_loop(0, k, body, xp)
    return _tc_linear(h[:n_nodes], W.T, b)
